# baseline (speedup 1.0000x reference)
import functools

import jax
import jax.numpy as jnp
from jax import lax
from jax.experimental import pallas as pl
from jax.experimental.pallas import tpu as pltpu
from jax.experimental.pallas import tpu_sc as plsc

NC, NS, L = 2, 16, 16
NW = NC * NS
H = 128
HB = H // L
CHUNK = 80

CP = 10240
TR = CP


def _proj_body(nx_ref, cx_ref, wn_ref, wc_ref, a_ref, b_ref):
    dn = (((1,), (1,)), ((), ()))
    a_ref[...] = lax.dot_general(nx_ref[...], wn_ref[...], dn,
                                 preferred_element_type=jnp.float32)
    b_ref[...] = lax.dot_general(cx_ref[...], wc_ref[...], dn,
                                 preferred_element_type=jnp.float32)


def _score_body(g_ref, v_ref, p_ref):
    t = jnp.tanh(g_ref[...])
    s = jnp.sum(t * v_ref[...], axis=1)
    p_ref[...] = jnp.exp(s)


def _final_body(f0_ref, f1_ref, f2_ref, f3_ref, d0_ref, d1_ref, o_ref):
    acc = f0_ref[0] + f1_ref[0] + f2_ref[0] + f3_ref[0]
    dpk = (jnp.sum(d0_ref[...], axis=(0, 1))
           + jnp.sum(d1_ref[...], axis=(0, 1)))
    n = acc.shape[0]
    rsel = (lax.broadcasted_iota(jnp.int32, (n, n // H), 0) // H
            == lax.broadcasted_iota(jnp.int32, (n, n // H), 1))
    den_rows = jax.lax.dot_general(rsel.astype(jnp.float32), dpk,
                                   (((1,), (0,)), ((), ())),
                                   preferred_element_type=jnp.float32)
    lsel = (lax.broadcasted_iota(jnp.int32, (n, H), 1)
            == lax.broadcasted_iota(jnp.int32, (n, H), 0) % H)
    den = jnp.sum(jnp.where(lsel, den_rows, 0.0), axis=1, keepdims=True)
    o_ref[...] = acc / jnp.maximum(den, 1e-9)


GC = 128


def _gather_add_body(src_hbm, dst_hbm, a_hbm, b_hbm, g_hbm,
                     si_v, di_v, g0_v, g1_v,
                     sa0, sa1, sb0, sb1, sw0, sw1):
    wid = lax.axis_index("s") * NC + lax.axis_index("c")
    epw = src_hbm.shape[0] // NW
    base = wid * epw
    n_full = epw // GC
    tail = epw - n_full * GC

    pltpu.sync_copy(src_hbm.at[pl.ds(base, epw)], si_v)
    pltpu.sync_copy(dst_hbm.at[pl.ds(base, epw)], di_v)

    def pair(k, _):
        c0 = 2 * k * GC
        c1 = c0 + GC
        a0 = pltpu.async_copy(a_hbm.at[si_v.at[pl.ds(c0, GC)]], g0_v, sa0)
        a1 = pltpu.async_copy(a_hbm.at[si_v.at[pl.ds(c1, GC)]], g1_v, sa1)
        a0.wait()
        b0 = pltpu.async_copy(b_hbm.at[di_v.at[pl.ds(c0, GC)]], g0_v, sb0,
                              add=True)
        a1.wait()
        b1 = pltpu.async_copy(b_hbm.at[di_v.at[pl.ds(c1, GC)]], g1_v, sb1,
                              add=True)
        b0.wait()
        w0 = pltpu.async_copy(g0_v, g_hbm.at[pl.ds(base + c0, GC)], sw0)
        b1.wait()
        w1 = pltpu.async_copy(g1_v, g_hbm.at[pl.ds(base + c1, GC)], sw1)
        w0.wait()
        w1.wait()
        return 0

    lax.fori_loop(0, n_full // 2, pair, 0)

    def one_chunk(off, size):
        pltpu.async_copy(a_hbm.at[si_v.at[pl.ds(off, size)]],
                         g0_v.at[pl.ds(0, size)], sa0).wait()
        pltpu.async_copy(b_hbm.at[di_v.at[pl.ds(off, size)]],
                         g0_v.at[pl.ds(0, size)], sb0, add=True).wait()
        pltpu.async_copy(g0_v.at[pl.ds(0, size)],
                         g_hbm.at[pl.ds(base + off, size)], sw0).wait()

    if n_full % 2:
        one_chunk((n_full - 1) * GC, GC)
    if tail:
        one_chunk(n_full * GC, tail)


SEC = 2000


def _accum_body(src_hbm, dst_hbm, x_hbm, p_hbm, z_hbm, zd_hbm,
                parts_hbm, dens_hbm,
                si_v, di_v, p_v, x0_v, x1_v, den_v,
                i0_v, i1_v, acc_sh,
                sg0, sg1, sf0, sf1):
    cid = lax.axis_index("c")
    sid = lax.axis_index("s")
    wid = sid * NC + cid
    epw = src_hbm.shape[0] // NW
    base = wid * epw
    rpt = acc_sh.shape[0] // NS

    pltpu.sync_copy(z_hbm, acc_sh.at[pl.ds(sid * rpt, rpt)])
    pltpu.sync_copy(zd_hbm, den_v)
    plsc.subcore_barrier()

    lane_iota = lax.iota(jnp.int32, L)

    def chunk_scatter(coff, x_v, i_v, sf):
        def group(k, _):
            eoff = coff + k * L
            ps = p_v[pl.ds(eoff, L)]
            di = di_v[pl.ds(eoff, L)]
            i_v[pl.ds(k * L, L)] = di
            dr = lax.shift_right_logical(di, 7)
            dc = lax.shift_right_logical(jnp.bitwise_and(di, H - 1), 4)
            dl = jnp.bitwise_and(di, L - 1)
            for j in range(L):
                e = k * L + j
                pb = lax.broadcast(ps[j], (L,))
                for h in range(HB):
                    x_v[e, pl.ds(h * L, L)] = x_v[e, pl.ds(h * L, L)] * pb
                oh = jnp.where(lane_iota == lax.broadcast(dl[j], (L,)),
                               pb, 0.0)
                sl = pl.ds(dc[j] * L, L)
                den_v[dr[j], sl] = den_v[dr[j], sl] + oh
            return 0

        lax.fori_loop(0, CHUNK // L, group, 0)
        return pltpu.async_copy(x_v, acc_sh.at[i_v], sf, add=True)

    def section(s, _):
        soff = base + s * SEC
        pltpu.sync_copy(src_hbm.at[pl.ds(soff, SEC)], si_v)
        pltpu.sync_copy(dst_hbm.at[pl.ds(soff, SEC)], di_v)
        pltpu.sync_copy(p_hbm.at[pl.ds(soff, SEC)], p_v)

        def pair(k, _):
            c0 = 2 * k * CHUNK
            c1 = c0 + CHUNK
            g0 = pltpu.async_copy(x_hbm.at[si_v.at[pl.ds(c0, CHUNK)]],
                                  x0_v, sg0)
            g1 = pltpu.async_copy(x_hbm.at[si_v.at[pl.ds(c1, CHUNK)]],
                                  x1_v, sg1)
            g0.wait()
            f0 = chunk_scatter(c0, x0_v, i0_v, sf0)
            g1.wait()
            f1 = chunk_scatter(c1, x1_v, i1_v, sf1)
            f0.wait()
            f1.wait()
            return 0

        n_chunks = SEC // CHUNK
        lax.fori_loop(0, n_chunks // 2, pair, 0)

        ct = (n_chunks - 1) * CHUNK
        pltpu.async_copy(x_hbm.at[si_v.at[pl.ds(ct, CHUNK)]],
                         x0_v, sg0).wait()
        chunk_scatter(ct, x0_v, i0_v, sf0).wait()
        return 0

    lax.fori_loop(0, epw // SEC, section, 0)

    plsc.subcore_barrier()
    pltpu.sync_copy(acc_sh.at[pl.ds(sid * rpt, rpt)],
                    parts_hbm.at[cid, pl.ds(sid * rpt, rpt)])
    pltpu.sync_copy(den_v, dens_hbm.at[cid, sid])


def kernel(news_x, company_x, edge_index, num_companies, W_news, W_company, v):
    N, Hd = news_x.shape
    C = company_x.shape[0]
    E = edge_index.shape[1]
    src = edge_index[0]
    dst = edge_index[1]

    rows_blk = 1000
    a, b = pl.pallas_call(
        _proj_body,
        grid=(N // rows_blk,),
        in_specs=[
            pl.BlockSpec((rows_blk, Hd), lambda i: (i, 0)),
            pl.BlockSpec((rows_blk, Hd), lambda i: (i, 0)),
            pl.BlockSpec((Hd, Hd), lambda i: (0, 0)),
            pl.BlockSpec((Hd, Hd), lambda i: (0, 0)),
        ],
        out_specs=[
            pl.BlockSpec((rows_blk, Hd), lambda i: (i, 0)),
            pl.BlockSpec((rows_blk, Hd), lambda i: (i, 0)),
        ],
        out_shape=[
            jax.ShapeDtypeStruct((N, Hd), jnp.float32),
            jax.ShapeDtypeStruct((C, Hd), jnp.float32),
        ],
    )(news_x, company_x, W_news, W_company)

    mesh = plsc.VectorSubcoreMesh(core_axis_name="c", subcore_axis_name="s",
                                  num_cores=NC, num_subcores=NS)

    def k2(src_b, dst_b):
        eb = src_b.shape[0]
        return pl.kernel(
            _gather_add_body,
            out_type=jax.ShapeDtypeStruct((eb, Hd), jnp.float32),
            mesh=mesh,
            scratch_types=[
                pltpu.VMEM((eb // NW,), jnp.int32),
                pltpu.VMEM((eb // NW,), jnp.int32),
                pltpu.VMEM((GC, Hd), jnp.float32),
                pltpu.VMEM((GC, Hd), jnp.float32),
            ] + [pltpu.SemaphoreType.DMA] * 6,
        )(src_b, dst_b, a, b)

    def k3(g_b):
        eb = g_b.shape[0]
        e_blk = 512
        return pl.pallas_call(
            _score_body,
            grid=(eb // e_blk,),
            in_specs=[
                pl.BlockSpec((e_blk, Hd), lambda i: (i, 0)),
                pl.BlockSpec((1, Hd), lambda i: (0, 0)),
            ],
            out_specs=pl.BlockSpec((e_blk,), lambda i: (i,)),
            out_shape=jax.ShapeDtypeStruct((eb,), jnp.float32),
        )(g_b, v)

    zeros_tile = jnp.zeros((TR // NS, Hd), jnp.float32)
    zeros_den = jnp.zeros((CP // Hd, Hd), jnp.float32)

    def k4(src_b, dst_b, p_b):
        return pl.kernel(
            _accum_body,
            out_type=(jax.ShapeDtypeStruct((NC, TR, Hd), jnp.float32),
                      jax.ShapeDtypeStruct((NC, NS, CP // Hd, Hd),
                                           jnp.float32)),
            mesh=mesh,
            scratch_types=[
                pltpu.VMEM((SEC,), jnp.int32),
                pltpu.VMEM((SEC,), jnp.int32),
                pltpu.VMEM((SEC,), jnp.float32),
                pltpu.VMEM((CHUNK, Hd), jnp.float32),
                pltpu.VMEM((CHUNK, Hd), jnp.float32),
                pltpu.VMEM((CP // Hd, Hd), jnp.float32),
                pltpu.VMEM((CHUNK,), jnp.int32),
                pltpu.VMEM((CHUNK,), jnp.int32),
                pltpu.VMEM_SHARED((TR, Hd), jnp.float32),
            ] + [pltpu.SemaphoreType.DMA] * 4,
        )(src_b, dst_b, news_x, p_b, zeros_tile, zeros_den)

    E0 = E * 3 // 5
    g0 = k2(src[:E0], dst[:E0])
    g1 = k2(src[E0:], dst[E0:])
    p0 = k3(g0)
    p1 = k3(g1)
    parts0, dens0 = k4(src[:E0], dst[:E0], p0)
    parts1, dens1 = k4(src[E0:], dst[E0:], p1)

    c_blk = 1024
    n_blk = (C + c_blk - 1) // c_blk
    fspec = [pl.BlockSpec((1, c_blk, Hd), lambda i, c=c: (c, i, 0))
             for c in (0, 1)]
    dspec = pl.BlockSpec((NC, NS, c_blk // Hd, Hd), lambda i: (0, 0, i, 0))
    out = pl.pallas_call(
        _final_body,
        grid=(n_blk,),
        in_specs=fspec + fspec + [dspec, dspec],
        out_specs=pl.BlockSpec((c_blk, Hd), lambda i: (i, 0)),
        out_shape=jax.ShapeDtypeStruct((C, Hd), jnp.float32),
    )(parts0, parts0, parts1, parts1, dens0, dens1)
    return out


# speedup vs baseline: 1.0166x; 1.0166x over previous
import functools

import jax
import jax.numpy as jnp
from jax import lax
from jax.experimental import pallas as pl
from jax.experimental.pallas import tpu as pltpu
from jax.experimental.pallas import tpu_sc as plsc

NC, NS, L = 2, 16, 16
NW = NC * NS
H = 128
HB = H // L
CHUNK = 80

CP = 10240
TR = CP


def _proj_body(nx_ref, cx_ref, wn_ref, wc_ref, a_ref, b_ref):
    dn = (((1,), (1,)), ((), ()))
    a_ref[...] = lax.dot_general(nx_ref[...], wn_ref[...], dn,
                                 preferred_element_type=jnp.float32)
    b_ref[...] = lax.dot_general(cx_ref[...], wc_ref[...], dn,
                                 preferred_element_type=jnp.float32)


def _score_body(g_ref, v_ref, p_ref):
    t = jnp.tanh(g_ref[...])
    s = jnp.sum(t * v_ref[...], axis=1)
    p_ref[...] = jnp.exp(s)


def _final_body(f0_ref, f1_ref, f2_ref, f3_ref, d0_ref, d1_ref, o_ref):
    acc = f0_ref[0] + f1_ref[0] + f2_ref[0] + f3_ref[0]
    dpk = (jnp.sum(d0_ref[...], axis=(0, 1))
           + jnp.sum(d1_ref[...], axis=(0, 1)))
    n = acc.shape[0]
    rsel = (lax.broadcasted_iota(jnp.int32, (n, n // H), 0) // H
            == lax.broadcasted_iota(jnp.int32, (n, n // H), 1))
    den_rows = jax.lax.dot_general(rsel.astype(jnp.float32), dpk,
                                   (((1,), (0,)), ((), ())),
                                   preferred_element_type=jnp.float32)
    lsel = (lax.broadcasted_iota(jnp.int32, (n, H), 1)
            == lax.broadcasted_iota(jnp.int32, (n, H), 0) % H)
    den = jnp.sum(jnp.where(lsel, den_rows, 0.0), axis=1, keepdims=True)
    o_ref[...] = acc / jnp.maximum(den, 1e-9)


GC = 128


def _gather_add_body(src_hbm, dst_hbm, a_hbm, b_hbm, g_hbm,
                     si_v, di_v, g0_v, g1_v, g2_v, g3_v,
                     sa0, sa1, sa2, sa3, sb0, sb1, sb2, sb3,
                     sw0, sw1, sw2, sw3):
    wid = lax.axis_index("s") * NC + lax.axis_index("c")
    epw = src_hbm.shape[0] // NW
    base = wid * epw
    n_full = epw // GC
    tail = epw - n_full * GC

    pltpu.sync_copy(src_hbm.at[pl.ds(base, epw)], si_v)
    pltpu.sync_copy(dst_hbm.at[pl.ds(base, epw)], di_v)

    bufs = (g0_v, g1_v, g2_v, g3_v)
    sas = (sa0, sa1, sa2, sa3)
    sbs = (sb0, sb1, sb2, sb3)
    sws = (sw0, sw1, sw2, sw3)

    def quad(k, _):
        c0 = 4 * k * GC
        avs = [pltpu.async_copy(a_hbm.at[si_v.at[pl.ds(c0 + q * GC, GC)]],
                                bufs[q], sas[q]) for q in range(4)]
        bvs = []
        for q in range(4):
            avs[q].wait()
            bvs.append(pltpu.async_copy(
                b_hbm.at[di_v.at[pl.ds(c0 + q * GC, GC)]], bufs[q], sbs[q],
                add=True))
        wvs = []
        for q in range(4):
            bvs[q].wait()
            wvs.append(pltpu.async_copy(
                bufs[q], g_hbm.at[pl.ds(base + c0 + q * GC, GC)], sws[q]))
        for q in range(4):
            wvs[q].wait()
        return 0

    lax.fori_loop(0, n_full // 4, quad, 0)

    def one_chunk(off, size, q):
        pltpu.async_copy(a_hbm.at[si_v.at[pl.ds(off, size)]],
                         bufs[q].at[pl.ds(0, size)], sas[q]).wait()
        pltpu.async_copy(b_hbm.at[di_v.at[pl.ds(off, size)]],
                         bufs[q].at[pl.ds(0, size)], sbs[q], add=True).wait()
        pltpu.async_copy(bufs[q].at[pl.ds(0, size)],
                         g_hbm.at[pl.ds(base + off, size)], sws[q]).wait()

    for i in range(n_full % 4):
        one_chunk((n_full - n_full % 4 + i) * GC, GC, i)
    if tail:
        one_chunk(n_full * GC, tail, 0)


SEC = 2000


def _accum_body(src_hbm, dst_hbm, x_hbm, p_hbm, z_hbm, zd_hbm,
                parts_hbm, dens_hbm,
                si_v, di_v, p_v, x0_v, x1_v, x2_v, den_v,
                i0_v, i1_v, i2_v, acc_sh,
                sg0, sg1, sg2, sf0, sf1, sf2):
    cid = lax.axis_index("c")
    sid = lax.axis_index("s")
    wid = sid * NC + cid
    epw = src_hbm.shape[0] // NW
    base = wid * epw
    rpt = acc_sh.shape[0] // NS

    pltpu.sync_copy(z_hbm, acc_sh.at[pl.ds(sid * rpt, rpt)])
    pltpu.sync_copy(zd_hbm, den_v)
    plsc.subcore_barrier()

    lane_iota = lax.iota(jnp.int32, L)

    def chunk_scatter(coff, x_v, i_v, sf):
        def group(k, _):
            eoff = coff + k * L
            ps = p_v[pl.ds(eoff, L)]
            di = di_v[pl.ds(eoff, L)]
            i_v[pl.ds(k * L, L)] = di
            dr = lax.shift_right_logical(di, 7)
            dc = lax.shift_right_logical(jnp.bitwise_and(di, H - 1), 4)
            dl = jnp.bitwise_and(di, L - 1)
            for j in range(L):
                e = k * L + j
                pb = lax.broadcast(ps[j], (L,))
                for h in range(HB):
                    x_v[e, pl.ds(h * L, L)] = x_v[e, pl.ds(h * L, L)] * pb
                oh = jnp.where(lane_iota == lax.broadcast(dl[j], (L,)),
                               pb, 0.0)
                sl = pl.ds(dc[j] * L, L)
                den_v[dr[j], sl] = den_v[dr[j], sl] + oh
            return 0

        lax.fori_loop(0, CHUNK // L, group, 0)
        return pltpu.async_copy(x_v, acc_sh.at[i_v], sf, add=True)

    def section(s, _):
        soff = base + s * SEC
        pltpu.sync_copy(src_hbm.at[pl.ds(soff, SEC)], si_v)
        pltpu.sync_copy(dst_hbm.at[pl.ds(soff, SEC)], di_v)
        pltpu.sync_copy(p_hbm.at[pl.ds(soff, SEC)], p_v)

        def triple(k, _):
            c0 = 3 * k * CHUNK
            c1 = c0 + CHUNK
            c2 = c1 + CHUNK
            g0 = pltpu.async_copy(x_hbm.at[si_v.at[pl.ds(c0, CHUNK)]],
                                  x0_v, sg0)
            g1 = pltpu.async_copy(x_hbm.at[si_v.at[pl.ds(c1, CHUNK)]],
                                  x1_v, sg1)
            g2 = pltpu.async_copy(x_hbm.at[si_v.at[pl.ds(c2, CHUNK)]],
                                  x2_v, sg2)
            g0.wait()
            f0 = chunk_scatter(c0, x0_v, i0_v, sf0)
            g1.wait()
            f1 = chunk_scatter(c1, x1_v, i1_v, sf1)
            g2.wait()
            f2 = chunk_scatter(c2, x2_v, i2_v, sf2)
            f0.wait()
            f1.wait()
            f2.wait()
            return 0

        n_chunks = SEC // CHUNK
        lax.fori_loop(0, n_chunks // 3, triple, 0)

        ct = (n_chunks - n_chunks % 3) * CHUNK
        pltpu.async_copy(x_hbm.at[si_v.at[pl.ds(ct, CHUNK)]],
                         x0_v, sg0).wait()
        chunk_scatter(ct, x0_v, i0_v, sf0).wait()
        return 0

    lax.fori_loop(0, epw // SEC, section, 0)

    plsc.subcore_barrier()
    pltpu.sync_copy(acc_sh.at[pl.ds(sid * rpt, rpt)],
                    parts_hbm.at[cid, pl.ds(sid * rpt, rpt)])
    pltpu.sync_copy(den_v, dens_hbm.at[cid, sid])


def kernel(news_x, company_x, edge_index, num_companies, W_news, W_company, v):
    N, Hd = news_x.shape
    C = company_x.shape[0]
    E = edge_index.shape[1]
    src = edge_index[0]
    dst = edge_index[1]

    rows_blk = 1000
    a, b = pl.pallas_call(
        _proj_body,
        grid=(N // rows_blk,),
        in_specs=[
            pl.BlockSpec((rows_blk, Hd), lambda i: (i, 0)),
            pl.BlockSpec((rows_blk, Hd), lambda i: (i, 0)),
            pl.BlockSpec((Hd, Hd), lambda i: (0, 0)),
            pl.BlockSpec((Hd, Hd), lambda i: (0, 0)),
        ],
        out_specs=[
            pl.BlockSpec((rows_blk, Hd), lambda i: (i, 0)),
            pl.BlockSpec((rows_blk, Hd), lambda i: (i, 0)),
        ],
        out_shape=[
            jax.ShapeDtypeStruct((N, Hd), jnp.float32),
            jax.ShapeDtypeStruct((C, Hd), jnp.float32),
        ],
    )(news_x, company_x, W_news, W_company)

    mesh = plsc.VectorSubcoreMesh(core_axis_name="c", subcore_axis_name="s",
                                  num_cores=NC, num_subcores=NS)

    def k2(src_b, dst_b):
        eb = src_b.shape[0]
        return pl.kernel(
            _gather_add_body,
            out_type=jax.ShapeDtypeStruct((eb, Hd), jnp.float32),
            mesh=mesh,
            scratch_types=[
                pltpu.VMEM((eb // NW,), jnp.int32),
                pltpu.VMEM((eb // NW,), jnp.int32),
                pltpu.VMEM((GC, Hd), jnp.float32),
                pltpu.VMEM((GC, Hd), jnp.float32),
                pltpu.VMEM((GC, Hd), jnp.float32),
                pltpu.VMEM((GC, Hd), jnp.float32),
            ] + [pltpu.SemaphoreType.DMA] * 12,
        )(src_b, dst_b, a, b)

    def k3(g_b):
        eb = g_b.shape[0]
        e_blk = 512
        return pl.pallas_call(
            _score_body,
            grid=(eb // e_blk,),
            in_specs=[
                pl.BlockSpec((e_blk, Hd), lambda i: (i, 0)),
                pl.BlockSpec((1, Hd), lambda i: (0, 0)),
            ],
            out_specs=pl.BlockSpec((e_blk,), lambda i: (i,)),
            out_shape=jax.ShapeDtypeStruct((eb,), jnp.float32),
        )(g_b, v)

    zeros_tile = jnp.zeros((TR // NS, Hd), jnp.float32)
    zeros_den = jnp.zeros((CP // Hd, Hd), jnp.float32)

    def k4(src_b, dst_b, p_b):
        return pl.kernel(
            _accum_body,
            out_type=(jax.ShapeDtypeStruct((NC, TR, Hd), jnp.float32),
                      jax.ShapeDtypeStruct((NC, NS, CP // Hd, Hd),
                                           jnp.float32)),
            mesh=mesh,
            scratch_types=[
                pltpu.VMEM((SEC,), jnp.int32),
                pltpu.VMEM((SEC,), jnp.int32),
                pltpu.VMEM((SEC,), jnp.float32),
                pltpu.VMEM((CHUNK, Hd), jnp.float32),
                pltpu.VMEM((CHUNK, Hd), jnp.float32),
                pltpu.VMEM((CHUNK, Hd), jnp.float32),
                pltpu.VMEM((CP // Hd, Hd), jnp.float32),
                pltpu.VMEM((CHUNK,), jnp.int32),
                pltpu.VMEM((CHUNK,), jnp.int32),
                pltpu.VMEM((CHUNK,), jnp.int32),
                pltpu.VMEM_SHARED((TR, Hd), jnp.float32),
            ] + [pltpu.SemaphoreType.DMA] * 6,
        )(src_b, dst_b, news_x, p_b, zeros_tile, zeros_den)

    E0 = E * 3 // 5
    g0 = k2(src[:E0], dst[:E0])
    g1 = k2(src[E0:], dst[E0:])
    p0 = k3(g0)
    p1 = k3(g1)
    parts0, dens0 = k4(src[:E0], dst[:E0], p0)
    parts1, dens1 = k4(src[E0:], dst[E0:], p1)

    c_blk = 1024
    n_blk = (C + c_blk - 1) // c_blk
    fspec = [pl.BlockSpec((1, c_blk, Hd), lambda i, c=c: (c, i, 0))
             for c in (0, 1)]
    dspec = pl.BlockSpec((NC, NS, c_blk // Hd, Hd), lambda i: (0, 0, i, 0))
    out = pl.pallas_call(
        _final_body,
        grid=(n_blk,),
        in_specs=fspec + fspec + [dspec, dspec],
        out_specs=pl.BlockSpec((c_blk, Hd), lambda i: (i, 0)),
        out_shape=jax.ShapeDtypeStruct((C, Hd), jnp.float32),
    )(parts0, parts0, parts1, parts1, dens0, dens1)
    return out


# speedup vs baseline: 1.0719x; 1.0544x over previous
import functools

import jax
import jax.numpy as jnp
from jax import lax
from jax.experimental import pallas as pl
from jax.experimental.pallas import tpu as pltpu
from jax.experimental.pallas import tpu_sc as plsc

NC, NS, L = 2, 16, 16
NW = NC * NS
H = 128
HB = H // L
CHUNK = 80

CP = 10240
TR = CP


def _proj_body(nx_ref, cx_ref, wn_ref, wc_ref, a_ref, b_ref):
    dn = (((1,), (1,)), ((), ()))
    a_ref[...] = lax.dot_general(nx_ref[...], wn_ref[...], dn,
                                 preferred_element_type=jnp.float32)
    b_ref[...] = lax.dot_general(cx_ref[...], wc_ref[...], dn,
                                 preferred_element_type=jnp.float32)


def _score_body(g_ref, v_ref, p_ref):
    t = jnp.tanh(g_ref[...])
    s = jnp.sum(t * v_ref[...], axis=1)
    p_ref[...] = jnp.exp(s)


def _final_body(f0_ref, f1_ref, f2_ref, f3_ref, f4_ref, f5_ref,
                d0_ref, d1_ref, d2_ref, o_ref):
    acc = (f0_ref[0] + f1_ref[0] + f2_ref[0]
           + f3_ref[0] + f4_ref[0] + f5_ref[0])
    dpk = (jnp.sum(d0_ref[...], axis=(0, 1))
           + jnp.sum(d1_ref[...], axis=(0, 1))
           + jnp.sum(d2_ref[...], axis=(0, 1)))
    n = acc.shape[0]
    rsel = (lax.broadcasted_iota(jnp.int32, (n, n // H), 0) // H
            == lax.broadcasted_iota(jnp.int32, (n, n // H), 1))
    den_rows = jax.lax.dot_general(rsel.astype(jnp.float32), dpk,
                                   (((1,), (0,)), ((), ())),
                                   preferred_element_type=jnp.float32)
    lsel = (lax.broadcasted_iota(jnp.int32, (n, H), 1)
            == lax.broadcasted_iota(jnp.int32, (n, H), 0) % H)
    den = jnp.sum(jnp.where(lsel, den_rows, 0.0), axis=1, keepdims=True)
    o_ref[...] = acc / jnp.maximum(den, 1e-9)


GC = 128


def _gather_add_body(src_hbm, dst_hbm, a_hbm, b_hbm, g_hbm,
                     si_v, di_v, g0_v, g1_v, g2_v, g3_v,
                     sa0, sa1, sa2, sa3, sb0, sb1, sb2, sb3,
                     sw0, sw1, sw2, sw3):
    wid = lax.axis_index("s") * NC + lax.axis_index("c")
    epw = src_hbm.shape[0] // NW
    base = wid * epw
    n_full = epw // GC
    tail = epw - n_full * GC

    pltpu.sync_copy(src_hbm.at[pl.ds(base, epw)], si_v)
    pltpu.sync_copy(dst_hbm.at[pl.ds(base, epw)], di_v)

    bufs = (g0_v, g1_v, g2_v, g3_v)
    sas = (sa0, sa1, sa2, sa3)
    sbs = (sb0, sb1, sb2, sb3)
    sws = (sw0, sw1, sw2, sw3)

    def quad(k, _):
        c0 = 4 * k * GC
        avs = [pltpu.async_copy(a_hbm.at[si_v.at[pl.ds(c0 + q * GC, GC)]],
                                bufs[q], sas[q]) for q in range(4)]
        bvs = []
        for q in range(4):
            avs[q].wait()
            bvs.append(pltpu.async_copy(
                b_hbm.at[di_v.at[pl.ds(c0 + q * GC, GC)]], bufs[q], sbs[q],
                add=True))
        wvs = []
        for q in range(4):
            bvs[q].wait()
            wvs.append(pltpu.async_copy(
                bufs[q], g_hbm.at[pl.ds(base + c0 + q * GC, GC)], sws[q]))
        for q in range(4):
            wvs[q].wait()
        return 0

    lax.fori_loop(0, n_full // 4, quad, 0)

    def one_chunk(off, size, q):
        pltpu.async_copy(a_hbm.at[si_v.at[pl.ds(off, size)]],
                         bufs[q].at[pl.ds(0, size)], sas[q]).wait()
        pltpu.async_copy(b_hbm.at[di_v.at[pl.ds(off, size)]],
                         bufs[q].at[pl.ds(0, size)], sbs[q], add=True).wait()
        pltpu.async_copy(bufs[q].at[pl.ds(0, size)],
                         g_hbm.at[pl.ds(base + off, size)], sws[q]).wait()

    for i in range(n_full % 4):
        one_chunk((n_full - n_full % 4 + i) * GC, GC, i)
    if tail:
        one_chunk(n_full * GC, tail, 0)


SEC = 2000


def _accum_body(src_hbm, dst_hbm, x_hbm, p_hbm, z_hbm, zd_hbm,
                parts_hbm, dens_hbm,
                si_v, di_v, p_v, x0_v, x1_v, x2_v, den_v,
                i0_v, i1_v, i2_v, acc_sh,
                sg0, sg1, sg2, sf0, sf1, sf2):
    cid = lax.axis_index("c")
    sid = lax.axis_index("s")
    wid = sid * NC + cid
    epw = src_hbm.shape[0] // NW
    base = wid * epw
    rpt = acc_sh.shape[0] // NS

    pltpu.sync_copy(z_hbm, acc_sh.at[pl.ds(sid * rpt, rpt)])
    pltpu.sync_copy(zd_hbm, den_v)
    plsc.subcore_barrier()

    lane_iota = lax.iota(jnp.int32, L)

    def chunk_scatter(coff, x_v, i_v, sf):
        def group(k, _):
            eoff = coff + k * L
            ps = p_v[pl.ds(eoff, L)]
            di = di_v[pl.ds(eoff, L)]
            i_v[pl.ds(k * L, L)] = di
            dr = lax.shift_right_logical(di, 7)
            dc = lax.shift_right_logical(jnp.bitwise_and(di, H - 1), 4)
            dl = jnp.bitwise_and(di, L - 1)
            for j in range(L):
                e = k * L + j
                pb = lax.broadcast(ps[j], (L,))
                for h in range(HB):
                    x_v[e, pl.ds(h * L, L)] = x_v[e, pl.ds(h * L, L)] * pb
                oh = jnp.where(lane_iota == lax.broadcast(dl[j], (L,)),
                               pb, 0.0)
                sl = pl.ds(dc[j] * L, L)
                den_v[dr[j], sl] = den_v[dr[j], sl] + oh
            return 0

        lax.fori_loop(0, CHUNK // L, group, 0)
        return pltpu.async_copy(x_v, acc_sh.at[i_v], sf, add=True)

    def section(s, _):
        soff = base + s * SEC
        pltpu.sync_copy(src_hbm.at[pl.ds(soff, SEC)], si_v)
        pltpu.sync_copy(dst_hbm.at[pl.ds(soff, SEC)], di_v)
        pltpu.sync_copy(p_hbm.at[pl.ds(soff, SEC)], p_v)

        def triple(k, _):
            c0 = 3 * k * CHUNK
            c1 = c0 + CHUNK
            c2 = c1 + CHUNK
            g0 = pltpu.async_copy(x_hbm.at[si_v.at[pl.ds(c0, CHUNK)]],
                                  x0_v, sg0)
            g1 = pltpu.async_copy(x_hbm.at[si_v.at[pl.ds(c1, CHUNK)]],
                                  x1_v, sg1)
            g2 = pltpu.async_copy(x_hbm.at[si_v.at[pl.ds(c2, CHUNK)]],
                                  x2_v, sg2)
            g0.wait()
            f0 = chunk_scatter(c0, x0_v, i0_v, sf0)
            g1.wait()
            f1 = chunk_scatter(c1, x1_v, i1_v, sf1)
            g2.wait()
            f2 = chunk_scatter(c2, x2_v, i2_v, sf2)
            f0.wait()
            f1.wait()
            f2.wait()
            return 0

        n_chunks = SEC // CHUNK
        lax.fori_loop(0, n_chunks // 3, triple, 0)

        ct = (n_chunks - n_chunks % 3) * CHUNK
        pltpu.async_copy(x_hbm.at[si_v.at[pl.ds(ct, CHUNK)]],
                         x0_v, sg0).wait()
        chunk_scatter(ct, x0_v, i0_v, sf0).wait()
        return 0

    lax.fori_loop(0, epw // SEC, section, 0)

    plsc.subcore_barrier()
    pltpu.sync_copy(acc_sh.at[pl.ds(sid * rpt, rpt)],
                    parts_hbm.at[cid, pl.ds(sid * rpt, rpt)])
    pltpu.sync_copy(den_v, dens_hbm.at[cid, sid])


def kernel(news_x, company_x, edge_index, num_companies, W_news, W_company, v):
    N, Hd = news_x.shape
    C = company_x.shape[0]
    E = edge_index.shape[1]
    src = edge_index[0]
    dst = edge_index[1]

    rows_blk = 1000
    a, b = pl.pallas_call(
        _proj_body,
        grid=(N // rows_blk,),
        in_specs=[
            pl.BlockSpec((rows_blk, Hd), lambda i: (i, 0)),
            pl.BlockSpec((rows_blk, Hd), lambda i: (i, 0)),
            pl.BlockSpec((Hd, Hd), lambda i: (0, 0)),
            pl.BlockSpec((Hd, Hd), lambda i: (0, 0)),
        ],
        out_specs=[
            pl.BlockSpec((rows_blk, Hd), lambda i: (i, 0)),
            pl.BlockSpec((rows_blk, Hd), lambda i: (i, 0)),
        ],
        out_shape=[
            jax.ShapeDtypeStruct((N, Hd), jnp.float32),
            jax.ShapeDtypeStruct((C, Hd), jnp.float32),
        ],
    )(news_x, company_x, W_news, W_company)

    mesh = plsc.VectorSubcoreMesh(core_axis_name="c", subcore_axis_name="s",
                                  num_cores=NC, num_subcores=NS)

    def k2(src_b, dst_b):
        eb = src_b.shape[0]
        return pl.kernel(
            _gather_add_body,
            out_type=jax.ShapeDtypeStruct((eb, Hd), jnp.float32),
            mesh=mesh,
            scratch_types=[
                pltpu.VMEM((eb // NW,), jnp.int32),
                pltpu.VMEM((eb // NW,), jnp.int32),
                pltpu.VMEM((GC, Hd), jnp.float32),
                pltpu.VMEM((GC, Hd), jnp.float32),
                pltpu.VMEM((GC, Hd), jnp.float32),
                pltpu.VMEM((GC, Hd), jnp.float32),
            ] + [pltpu.SemaphoreType.DMA] * 12,
        )(src_b, dst_b, a, b)

    def k3(g_b):
        eb = g_b.shape[0]
        e_blk = 512
        return pl.pallas_call(
            _score_body,
            grid=(eb // e_blk,),
            in_specs=[
                pl.BlockSpec((e_blk, Hd), lambda i: (i, 0)),
                pl.BlockSpec((1, Hd), lambda i: (0, 0)),
            ],
            out_specs=pl.BlockSpec((e_blk,), lambda i: (i,)),
            out_shape=jax.ShapeDtypeStruct((eb,), jnp.float32),
        )(g_b, v)

    zeros_tile = jnp.zeros((TR // NS, Hd), jnp.float32)
    zeros_den = jnp.zeros((CP // Hd, Hd), jnp.float32)

    def k4(src_b, dst_b, p_b):
        return pl.kernel(
            _accum_body,
            out_type=(jax.ShapeDtypeStruct((NC, TR, Hd), jnp.float32),
                      jax.ShapeDtypeStruct((NC, NS, CP // Hd, Hd),
                                           jnp.float32)),
            mesh=mesh,
            scratch_types=[
                pltpu.VMEM((SEC,), jnp.int32),
                pltpu.VMEM((SEC,), jnp.int32),
                pltpu.VMEM((SEC,), jnp.float32),
                pltpu.VMEM((CHUNK, Hd), jnp.float32),
                pltpu.VMEM((CHUNK, Hd), jnp.float32),
                pltpu.VMEM((CHUNK, Hd), jnp.float32),
                pltpu.VMEM((CP // Hd, Hd), jnp.float32),
                pltpu.VMEM((CHUNK,), jnp.int32),
                pltpu.VMEM((CHUNK,), jnp.int32),
                pltpu.VMEM((CHUNK,), jnp.int32),
                pltpu.VMEM_SHARED((TR, Hd), jnp.float32),
            ] + [pltpu.SemaphoreType.DMA] * 6,
        )(src_b, dst_b, news_x, p_b, zeros_tile, zeros_den)

    cuts = (0, 2 * E // 5, 4 * E // 5, E)
    bs = [(src[cuts[i]:cuts[i + 1]], dst[cuts[i]:cuts[i + 1]])
          for i in range(3)]
    gs = [k2(s, d) for s, d in bs]
    ps = [k3(g) for g in gs]
    pd = [k4(s, d, p) for (s, d), p in zip(bs, ps)]

    c_blk = 1024
    n_blk = (C + c_blk - 1) // c_blk
    fspec = [pl.BlockSpec((1, c_blk, Hd), lambda i, c=c: (c, i, 0))
             for c in (0, 1)]
    dspec = pl.BlockSpec((NC, NS, c_blk // Hd, Hd), lambda i: (0, 0, i, 0))
    out = pl.pallas_call(
        _final_body,
        grid=(n_blk,),
        in_specs=fspec * 3 + [dspec] * 3,
        out_specs=pl.BlockSpec((c_blk, Hd), lambda i: (i, 0)),
        out_shape=jax.ShapeDtypeStruct((C, Hd), jnp.float32),
    )(pd[0][0], pd[0][0], pd[1][0], pd[1][0], pd[2][0], pd[2][0],
      pd[0][1], pd[1][1], pd[2][1])
    return out
